# baseline (speedup 1.0000x reference)
import functools

import numpy as np
import jax
import jax.numpy as jnp
from jax.experimental import pallas as pl
from jax.experimental.pallas import tpu as pltpu

RADIUS = 9.0
BI = 32
BJ = 512
DCUT = 25.0


def _group_norm(x, num_groups, gamma, beta, eps=1e-05):
    n, c = x.shape
    g = x.T.reshape(num_groups, (c // num_groups) * n)
    mean = g.mean(axis=1, keepdims=True)
    var = g.var(axis=1, keepdims=True)
    g = (g - mean) * jax.lax.rsqrt(var + eps)
    return g.reshape(c, n).T * gamma[None, :] + beta[None, :]


def _pairwise_kernel(nch_ref, xi_ref, ni_ref, m_ref, ci_ref, crows_ref,
                     w2t_ref, out_ref, acc_ref, *, cuts, h_ch, n):
    g = pl.program_id(0)
    nch = nch_ref[g]
    xi = [xi_ref[:, d:d + 1] for d in range(3)]
    ni = [ni_ref[:, d:d + 1] for d in range(3)]
    nq = BJ // 128
    acc_ref[...] = jnp.zeros(acc_ref.shape, jnp.float32)

    for t in range(n // BJ):
        @pl.when(t < nch)
        def _(t=t):
            off = t * BJ
            pj = [crows_ref[d:d + 1, off:off + BJ] for d in range(3)]
            nj = [crows_ref[3 + d:4 + d, off:off + BJ] for d in range(3)]
            dx = pj[0] - xi[0]
            dy = pj[1] - xi[1]
            dz = pj[2] - xi[2]
            r2 = dx * dx + dy * dy + dz * dz
            dot = ni[0] * nj[0] + ni[1] * nj[1] + ni[2] * nj[2]
            t2 = 2.0 - dot
            w = jnp.exp(-(r2 * (t2 * t2)))
            x1 = []
            for c in range(cuts):
                z = (m_ref[:, 3 * c:3 * c + 1] * pj[0]
                     + m_ref[:, 3 * c + 1:3 * c + 2] * pj[1]
                     + m_ref[:, 3 * c + 2:3 * c + 3] * pj[2]
                     + ci_ref[:, c:c + 1])
                x1.append(jnp.maximum(z, 0.0))
            for h in range(h_ch):
                z = w2t_ref[cuts:cuts + 1, h:h + 1]
                for c in range(cuts):
                    z = z + w2t_ref[c:c + 1, h:h + 1] * x1[c]
                zr = jnp.maximum(z, 0.0)
                fh = crows_ref[6 + h:7 + h, off:off + BJ]
                p = w * zr * fh
                s = p[:, 0:128]
                for q in range(1, nq):
                    s = s + p[:, 128 * q:128 * (q + 1)]
                acc_ref[:, 128 * h:128 * (h + 1)] += s

    out_ref[...] = jnp.concatenate(
        [jnp.sum(acc_ref[:, 128 * h:128 * (h + 1)], axis=1, keepdims=True)
         for h in range(h_ch)], axis=1)


def _compact_columns(pts_s, normals):
    n = pts_s.shape[0]
    pj2 = jnp.sum(pts_s * pts_s, axis=1)
    r2 = pj2[:, None] + pj2[None, :] - 2.0 * (pts_s @ pts_s.T)
    nd = normals @ normals.T
    t = 2.0 - nd
    d2 = jnp.maximum(r2, 0.0) * (t * t)
    far = (d2.reshape(n // BI, BI, n).min(axis=1) > DCUT)
    idx = jnp.argsort(far.astype(jnp.int32), axis=1,
                      stable=True).astype(jnp.int32)
    cnt = jnp.sum(~far, axis=1)
    nch = ((cnt + BJ - 1) // BJ).astype(jnp.int32)
    return idx, nch


def _pairwise_conv(pts_s, nuv, normals, f, p, idx, nch):
    n = pts_s.shape[0]
    cuts = p['conv_w1'].shape[0]
    h_ch = p['conv_w2'].shape[0]
    m = jnp.einsum('ck,ikd->icd', p['conv_w1'], nuv).reshape(n, 3 * cuts)
    ci = p['conv_b1'][None, :] - jnp.einsum('icd,id->ic',
                                            m.reshape(n, cuts, 3), pts_s)
    rows = jnp.concatenate(
        [pts_s.T, normals.T, f.T,
         jnp.zeros((2, n), jnp.float32)], axis=0)
    nrows = 6 + h_ch + 2
    crows = jnp.take(rows, idx.reshape(-1), axis=1)
    crows = crows.reshape(nrows, n // BI, n).transpose(1, 0, 2)
    crows = crows.reshape((n // BI) * nrows, n)
    w2t = jnp.concatenate([p['conv_w2'].T, p['conv_b2'][None, :]], axis=0)
    w2t = jnp.pad(w2t, ((0, 16 - w2t.shape[0]), (0, 0)))

    kern = functools.partial(_pairwise_kernel, cuts=cuts, h_ch=h_ch, n=n)
    grid_spec = pltpu.PrefetchScalarGridSpec(
        num_scalar_prefetch=1,
        grid=(n // BI,),
        scratch_shapes=[pltpu.VMEM((BI, 128 * h_ch), jnp.float32)],
        in_specs=[
            pl.BlockSpec((BI, 3), lambda g, *_: (g, 0)),
            pl.BlockSpec((BI, 3), lambda g, *_: (g, 0)),
            pl.BlockSpec((BI, 3 * cuts), lambda g, *_: (g, 0)),
            pl.BlockSpec((BI, cuts), lambda g, *_: (g, 0)),
            pl.BlockSpec((nrows, n), lambda g, *_: (g, 0)),
            pl.BlockSpec((16, h_ch), lambda g, *_: (0, 0)),
        ],
        out_specs=pl.BlockSpec((BI, h_ch), lambda g, *_: (g, 0)),
    )
    return pl.pallas_call(
        kern,
        grid_spec=grid_spec,
        out_shape=jax.ShapeDtypeStruct((n, h_ch), jnp.float32),
    )(nch, pts_s, normals, m, ci, crows, w2t)


def _leaky(x, slope=0.2):
    return jnp.where(x >= 0, x, slope * x)


def _conv_forward(pts_s, nuv, normals, feats, p, idx, nch, perm, inv):
    f = _leaky(feats @ p['w_in1'].T + p['b_in1'])
    f = _leaky(f @ p['w_in2'].T + p['b_in2'])
    f = _group_norm(f, 4, p['gn_in_w'], p['gn_in_b'])
    out = _pairwise_conv(pts_s, nuv, normals, f[perm], p, idx, nch)[inv]
    o = _leaky(out @ p['w_out1'].T + p['b_out1'])
    o = _leaky(o @ p['w_out2'].T + p['b_out2'])
    return _group_norm(o, 4, p['gn_out_w'], p['gn_out_b'])


def _morton_perm(pts):
    lo = pts.min(axis=0)
    hi = pts.max(axis=0)
    q = jnp.clip((pts - lo) / jnp.maximum(hi - lo, 1e-9) * 1023.0, 0.0, 1023.0)
    q = q.astype(jnp.uint32)

    def spread(x):
        x = (x | (x << 16)) & jnp.uint32(0x030000FF)
        x = (x | (x << 8)) & jnp.uint32(0x0300F00F)
        x = (x | (x << 4)) & jnp.uint32(0x030C30C3)
        x = (x | (x << 2)) & jnp.uint32(0x09249249)
        return x

    code = (spread(q[:, 0]) << 2) | (spread(q[:, 1]) << 1) | spread(q[:, 2])
    return jnp.argsort(code)


def kernel(features, points, nuv, params):
    pts_s = points / (np.sqrt(2.0) * RADIUS)
    perm = _morton_perm(pts_s)
    inv = jnp.argsort(perm)
    pts_p = pts_s[perm]
    nuv_p = nuv[perm]
    normals_p = nuv_p[:, 0, :]
    idx, nch = _compact_columns(pts_p, normals_p)
    x = features
    i = 0
    while ('layer%d' % i) in params:
        p = params['layer%d' % i]
        xi = _conv_forward(pts_p, nuv_p, normals_p, x, p, idx, nch, perm, inv)
        xi = jnp.maximum(xi @ p['ll_w1'].T + p['ll_b1'], 0.0) @ p['ll_w2'].T \
            + p['ll_b2']
        x = x @ p['lt_w'].T + p['lt_b']
        x = x + xi
        i += 1
    return x


# speedup vs baseline: 2.2899x; 2.2899x over previous
import functools

import numpy as np
import jax
import jax.numpy as jnp
from jax.experimental import pallas as pl

RADIUS = 9.0
BI = 32


def _group_norm(x, num_groups, gamma, beta, eps=1e-05):
    n, c = x.shape
    g = x.T.reshape(num_groups, (c // num_groups) * n)
    mean = g.mean(axis=1, keepdims=True)
    var = g.var(axis=1, keepdims=True)
    g = (g - mean) * jax.lax.rsqrt(var + eps)
    return g.reshape(c, n).T * gamma[None, :] + beta[None, :]


def _b16(x):
    return x.astype(jnp.bfloat16).astype(jnp.float32)


def _pairwise_kernel(xi_ref, ni_ref, nv_ref, rows_ref, wt_ref, out_ref,
                     *, cuts, h_ch):
    pj = [rows_ref[d:d + 1, :] for d in range(3)]
    njb = [rows_ref[3 + d:4 + d, :] for d in range(3)]
    dx = pj[0] - xi_ref[:, 0:1]
    dy = pj[1] - xi_ref[:, 1:2]
    dz = pj[2] - xi_ref[:, 2:3]
    r2 = dx * dx + dy * dy + dz * dz
    dot = (ni_ref[:, 0:1] * njb[0] + ni_ref[:, 1:2] * njb[1]
           + ni_ref[:, 2:3] * njb[2])
    t = 2.0 - dot
    w = jnp.exp(-(r2 * (t * t)))
    dxb = _b16(dx)
    dyb = _b16(dy)
    dzb = _b16(dz)
    xk = []
    for k in range(3):
        xk.append(_b16(nv_ref[:, 3 * k:3 * k + 1] * dxb
                       + nv_ref[:, 3 * k + 1:3 * k + 2] * dyb
                       + nv_ref[:, 3 * k + 2:3 * k + 3] * dzb))
    x1 = []
    for c in range(cuts):
        z = (wt_ref[17 + c:18 + c, 0:1] * xk[0]
             + wt_ref[17 + c:18 + c, 1:2] * xk[1]
             + wt_ref[17 + c:18 + c, 2:3] * xk[2]
             + wt_ref[16:17, c:c + 1])
        x1.append(_b16(jnp.maximum(z, 0.0)))
    outs = []
    for h in range(h_ch):
        z = wt_ref[cuts:cuts + 1, h:h + 1]
        for c in range(cuts):
            z = z + wt_ref[c:c + 1, h:h + 1] * x1[c]
        zr = jnp.maximum(z, 0.0)
        fh = rows_ref[6 + h:7 + h, :]
        outs.append(jnp.sum(w * zr * fh, axis=1, keepdims=True))
    out_ref[...] = jnp.concatenate(outs, axis=1)


def _pairwise_conv(pts_s, nuv, normals, f, p):
    n = pts_s.shape[0]
    cuts = p['conv_w1'].shape[0]
    h_ch = p['conv_w2'].shape[0]
    nb = _b16(normals)
    nvb = _b16(nuv).reshape(n, 9)
    rows = jnp.concatenate(
        [pts_s.T, nb.T, f.T,
         jnp.zeros((2, n), jnp.float32)], axis=0)
    wt = jnp.zeros((17 + cuts, h_ch), jnp.float32)
    wt = wt.at[0:cuts, :].set(_b16(p['conv_w2'].T))
    wt = wt.at[cuts, :].set(p['conv_b2'])
    wt = wt.at[16, 0:cuts].set(p['conv_b1'])
    wt = wt.at[17:17 + cuts, 0:3].set(_b16(p['conv_w1']))

    kern = functools.partial(_pairwise_kernel, cuts=cuts, h_ch=h_ch)
    grid = (n // BI,)
    return pl.pallas_call(
        kern,
        grid=grid,
        in_specs=[
            pl.BlockSpec((BI, 3), lambda g: (g, 0)),
            pl.BlockSpec((BI, 3), lambda g: (g, 0)),
            pl.BlockSpec((BI, 9), lambda g: (g, 0)),
            pl.BlockSpec((6 + h_ch + 2, n), lambda g: (0, 0)),
            pl.BlockSpec((17 + cuts, h_ch), lambda g: (0, 0)),
        ],
        out_specs=pl.BlockSpec((BI, h_ch), lambda g: (g, 0)),
        out_shape=jax.ShapeDtypeStruct((n, h_ch), jnp.float32),
    )(pts_s, nb, nvb, rows, wt)


def _leaky(x, slope=0.2):
    return jnp.where(x >= 0, x, slope * x)


def _conv_forward(pts_s, nuv, normals, feats, p):
    f = _leaky(feats @ p['w_in1'].T + p['b_in1'])
    f = _leaky(f @ p['w_in2'].T + p['b_in2'])
    f = _group_norm(f, 4, p['gn_in_w'], p['gn_in_b'])
    out = _pairwise_conv(pts_s, nuv, normals, f, p)
    o = _leaky(out @ p['w_out1'].T + p['b_out1'])
    o = _leaky(o @ p['w_out2'].T + p['b_out2'])
    return _group_norm(o, 4, p['gn_out_w'], p['gn_out_b'])


def kernel(features, points, nuv, params):
    pts_s = points / (np.sqrt(2.0) * RADIUS)
    normals = nuv[:, 0, :]
    x = features
    i = 0
    while ('layer%d' % i) in params:
        p = params['layer%d' % i]
        xi = _conv_forward(pts_s, nuv, normals, x, p)
        xi = jnp.maximum(xi @ p['ll_w1'].T + p['ll_b1'], 0.0) @ p['ll_w2'].T \
            + p['ll_b2']
        x = x @ p['lt_w'].T + p['lt_b']
        x = x + xi
        i += 1
    return x
